# Initial kernel scaffold; baseline (speedup 1.0000x reference)
import functools

import jax
import jax.numpy as jnp
from jax import lax
from jax.experimental import pallas as pl
from jax.experimental.pallas import tpu as pltpu

N = 1024
D = 128
E = 32768
K = 257
KPAD = 512

_INTERPRET = False


def _dense_body(adj_ref, x_ref, w_ref, b_ref, cx_ref, cadj_ref, s_ref, topk_ref):
    adj = adj_ref[...]
    x = x_ref[...]
    W = w_ref[...]
    b = b_ref[0]
    ones_col = jnp.ones((N, 1), jnp.float32)
    HI = lax.Precision.HIGHEST

    h = jnp.dot(x, W, precision=HI)
    colsum = lax.dot_general(adj, ones_col, (((0,), (0,)), ((), ())),
                             precision=HI)
    dinv = lax.rsqrt(colsum + 1.0)
    vh = dinv * h
    u = lax.dot_general(adj, vh, (((0,), (0,)), ((), ())), precision=HI)
    z = dinv * u + dinv * dinv * h
    alpha = jax.nn.sigmoid(jnp.square(z + b))

    alpha_rowb = lax.dot_general(ones_col, alpha, (((1,), (1,)), ((), ())),
                                 precision=HI)
    ii = lax.broadcasted_iota(jnp.int32, (N, N), 0)
    jj = lax.broadcasted_iota(jnp.int32, (N, N), 1)
    beats = (alpha_rowb > alpha) | ((alpha_rowb == alpha) & (jj < ii))
    rank = jnp.sum(jnp.where(beats, 1.0, 0.0), axis=1, keepdims=True)
    cut = jnp.sum(jnp.where(rank == (K - 1.0), alpha, 0.0))

    rowsum = lax.dot_general(adj, ones_col, (((1,), (0,)), ((), ())),
                             precision=HI)
    dinvr = lax.rsqrt(rowsum + 1.0)
    dj_rowb = lax.dot_general(ones_col, dinvr, (((1,), (1,)), ((), ())),
                              precision=HI)
    eye = jnp.where(ii == jj, 1.0, 0.0)
    di_col = jnp.where(rowsum > 0.0, dinvr, 0.0)
    norm_adj = di_col * (adj + eye) * dj_rowb
    cuta_rowb = jax.nn.relu(alpha_rowb + 1e-7 - cut)
    s0 = norm_adj * cuta_rowb
    rs = jnp.sum(jnp.abs(s0), axis=1, keepdims=True)
    S = s0 / jnp.maximum(rs, 1e-12)
    s_ref[...] = S

    cx_ref[...] = lax.dot_general(S, x, (((0,), (0,)), ((), ())))
    t1 = jnp.dot(adj, S)
    cadj = lax.dot_general(S, t1, (((0,), (0,)), ((), ())))
    cadj_ref[...] = jnp.floor(cadj * 10000.0) / 10000.0

    iota_k = lax.broadcasted_iota(jnp.int32, (N, KPAD), 1)
    ii_k = lax.broadcasted_iota(jnp.int32, (N, KPAD), 0)
    rank_i = rank.astype(jnp.int32)
    topk_ref[...] = jnp.sum(jnp.where(iota_k == rank_i, ii_k, 0),
                            axis=0, keepdims=True)


_dense_call = functools.partial(
    pl.pallas_call,
    out_shape=[
        jax.ShapeDtypeStruct((N, D), jnp.float32),
        jax.ShapeDtypeStruct((N, N), jnp.float32),
        jax.ShapeDtypeStruct((N, N), jnp.float32),
        jax.ShapeDtypeStruct((1, KPAD), jnp.int32),
    ],
    in_specs=[
        pl.BlockSpec(memory_space=pltpu.VMEM),
        pl.BlockSpec(memory_space=pltpu.VMEM),
        pl.BlockSpec(memory_space=pltpu.VMEM),
        pl.BlockSpec(memory_space=pltpu.SMEM),
    ],
    interpret=_INTERPRET,
)


def kernel(x, edge_index, edge_attr, W, b):
    ew = edge_attr[:, 0]
    src = edge_index[0]
    dst = edge_index[1]
    adj = jnp.zeros((N, N), jnp.float32).at[src, dst].add(ew)
    cx, cadj, S, topk = _dense_call(_dense_body)(adj, x, W, b)
    return cx, cadj, S, topk[0, :K]


# speedup vs baseline: 5.5503x; 5.5503x over previous
import functools

import jax
import jax.numpy as jnp
from jax import lax
from jax.experimental import pallas as pl
from jax.experimental.pallas import tpu as pltpu

N = 1024
D = 128
E = 32768
K = 257
KPAD = 512

_INTERPRET = False


def _dense_body(adj_ref, x_ref, w_ref, b_ref, cx_ref, cadj_ref, s_ref, topk_ref):
    adj = adj_ref[...]
    x = x_ref[...]
    W = w_ref[...]
    b = b_ref[0]
    ones_col = jnp.ones((N, 1), jnp.float32)
    HI = lax.Precision.HIGHEST

    h = jnp.dot(x, W)
    colsum = lax.dot_general(adj, ones_col, (((0,), (0,)), ((), ())),
                             precision=HI)
    dinv = lax.rsqrt(colsum + 1.0)
    vh = dinv * h
    u = lax.dot_general(adj, vh, (((0,), (0,)), ((), ())), precision=HI)
    z = dinv * u + dinv * dinv * h
    alpha = jax.nn.sigmoid(jnp.square(z + b))

    alpha_rowb = lax.dot_general(ones_col, alpha, (((1,), (1,)), ((), ())),
                                 precision=HI)
    ii = lax.broadcasted_iota(jnp.int32, (N, N), 0)
    jj = lax.broadcasted_iota(jnp.int32, (N, N), 1)
    beats = (alpha_rowb > alpha) | ((alpha_rowb == alpha) & (jj < ii))
    rank = jnp.sum(jnp.where(beats, 1.0, 0.0), axis=1, keepdims=True)
    cut = jnp.sum(jnp.where(rank == (K - 1.0), alpha, 0.0))

    rowsum = lax.dot_general(adj, ones_col, (((1,), (0,)), ((), ())),
                             precision=HI)
    dinvr = lax.rsqrt(rowsum + 1.0)
    dj_rowb = lax.dot_general(ones_col, dinvr, (((1,), (1,)), ((), ())),
                              precision=HI)
    eye = jnp.where(ii == jj, 1.0, 0.0)
    di_col = jnp.where(rowsum > 0.0, dinvr, 0.0)
    norm_adj = di_col * (adj + eye) * dj_rowb
    cuta_rowb = jax.nn.relu(alpha_rowb + 1e-7 - cut)
    s0 = norm_adj * cuta_rowb
    rs = jnp.sum(jnp.abs(s0), axis=1, keepdims=True)
    S = s0 / jnp.maximum(rs, 1e-12)
    s_ref[...] = S

    cx_ref[...] = lax.dot_general(S, x, (((0,), (0,)), ((), ())))
    t1 = jnp.dot(adj, S)
    cadj = lax.dot_general(S, t1, (((0,), (0,)), ((), ())))
    cadj_ref[...] = jnp.floor(cadj * 10000.0) / 10000.0

    iota_k = lax.broadcasted_iota(jnp.int32, (N, KPAD), 1)
    ii_k = lax.broadcasted_iota(jnp.int32, (N, KPAD), 0)
    rank_i = rank.astype(jnp.int32)
    topk_ref[...] = jnp.sum(jnp.where(iota_k == rank_i, ii_k, 0),
                            axis=0, keepdims=True)


_dense_call = functools.partial(
    pl.pallas_call,
    out_shape=[
        jax.ShapeDtypeStruct((N, D), jnp.float32),
        jax.ShapeDtypeStruct((N, N), jnp.float32),
        jax.ShapeDtypeStruct((N, N), jnp.float32),
        jax.ShapeDtypeStruct((1, KPAD), jnp.int32),
    ],
    in_specs=[
        pl.BlockSpec(memory_space=pltpu.VMEM),
        pl.BlockSpec(memory_space=pltpu.VMEM),
        pl.BlockSpec(memory_space=pltpu.VMEM),
        pl.BlockSpec(memory_space=pltpu.SMEM),
    ],
    interpret=_INTERPRET,
)


def kernel(x, edge_index, edge_attr, W, b):
    ew = edge_attr[:, 0]
    src = edge_index[0]
    dst = edge_index[1]
    adj = jnp.zeros((N, N), jnp.float32).at[src, dst].add(ew)
    cx, cadj, S, topk = _dense_call(_dense_body)(adj, x, W, b)
    return cx, cadj, S, topk[0, :K]


# speedup vs baseline: 6.4332x; 1.1591x over previous
import functools

import jax
import jax.numpy as jnp
from jax import lax
from jax.experimental import pallas as pl
from jax.experimental.pallas import tpu as pltpu
from jax.experimental.pallas import tpu_sc as plsc

N = 1024
D = 128
E = 32768
K = 257
KPAD = 512

_INTERPRET = False

_NC = 2
_NS = 16
_L = 16
_NW = _NC * _NS
_EPW = E // _NW
_CH = 128
_NCH = _EPW // _CH
_SL = (N * N) // _NS
_ZB = 8192


def _sc_adj_body(src_hbm, dst_hbm, ew_hbm, out_hbm,
                 src_v, dst_v, ew_v, idx_v, z_v, adj_sh, sem):
    c = lax.axis_index("c")
    s = lax.axis_index("s")
    wid = c * _NS + s
    ebase = wid * _EPW

    pltpu.sync_copy(src_hbm.at[pl.ds(ebase, _EPW)], src_v)
    pltpu.sync_copy(dst_hbm.at[pl.ds(ebase, _EPW)], dst_v)
    pltpu.sync_copy(ew_hbm.at[pl.ds(wid * _NCH, _NCH)], ew_v)

    def _zbody(i, carry):
        z_v[pl.ds(i * _L, _L)] = jnp.zeros((_L,), jnp.float32)
        return carry
    lax.fori_loop(0, _ZB // _L, _zbody, 0)
    for m in range(_SL // _ZB):
        pltpu.sync_copy(z_v, adj_sh.at[pl.ds(s * _SL + m * _ZB, _ZB)])

    for k in range(_EPW // _L):
        sv = src_v[pl.ds(k * _L, _L)]
        dv = dst_v[pl.ds(k * _L, _L)]
        idx_v[k // (_CH // _L), pl.ds((k % (_CH // _L)) * _L, _L)] = sv * N + dv

    plsc.subcore_barrier()

    copies = [
        pltpu.async_copy(ew_v.at[j], adj_sh.at[idx_v.at[j]], sem, add=True)
        for j in range(_NCH)
    ]
    for cp in copies:
        cp.wait()

    plsc.subcore_barrier()

    pltpu.sync_copy(adj_sh.at[pl.ds(s * _SL, _SL)],
                    out_hbm.at[c, pl.ds(s * _SL, _SL)])


_sc_adj_call = functools.partial(
    pl.kernel,
    out_type=jax.ShapeDtypeStruct((_NC, N * N), jnp.float32),
    mesh=plsc.VectorSubcoreMesh(core_axis_name="c", subcore_axis_name="s",
                                num_cores=_NC, num_subcores=_NS),
    scratch_types=[
        pltpu.VMEM((_EPW,), jnp.int32),
        pltpu.VMEM((_EPW,), jnp.int32),
        pltpu.VMEM((_NCH, _CH), jnp.float32),
        pltpu.VMEM((_NCH, _CH), jnp.int32),
        pltpu.VMEM((_ZB,), jnp.float32),
        pltpu.VMEM_SHARED((N * N,), jnp.float32),
        pltpu.SemaphoreType.DMA,
    ],
)


def _dense_body(a0_ref, a1_ref, x_ref, w_ref, b_ref,
                cx_ref, cadj_ref, s_ref, topk_ref):
    adj = a0_ref[...] + a1_ref[...]
    x = x_ref[...]
    W = w_ref[...]
    b = b_ref[0]
    ones_col = jnp.ones((N, 1), jnp.float32)
    HI = lax.Precision.HIGHEST

    h = jnp.dot(x, W)
    colsum = lax.dot_general(adj, ones_col, (((0,), (0,)), ((), ())),
                             precision=HI)
    dinv = lax.rsqrt(colsum + 1.0)
    vh = dinv * h
    u = lax.dot_general(adj, vh, (((0,), (0,)), ((), ())), precision=HI)
    z = dinv * u + dinv * dinv * h
    alpha = jax.nn.sigmoid(jnp.square(z + b))

    alpha_rowb = lax.dot_general(ones_col, alpha, (((1,), (1,)), ((), ())),
                                 precision=HI)
    ii = lax.broadcasted_iota(jnp.int32, (N, N), 0)
    jj = lax.broadcasted_iota(jnp.int32, (N, N), 1)
    beats = (alpha_rowb > alpha) | ((alpha_rowb == alpha) & (jj < ii))
    rank = jnp.sum(jnp.where(beats, 1.0, 0.0), axis=1, keepdims=True)
    cut = jnp.sum(jnp.where(rank == (K - 1.0), alpha, 0.0))

    rowsum = lax.dot_general(adj, ones_col, (((1,), (0,)), ((), ())),
                             precision=HI)
    dinvr = lax.rsqrt(rowsum + 1.0)
    dj_rowb = lax.dot_general(ones_col, dinvr, (((1,), (1,)), ((), ())),
                              precision=HI)
    eye = jnp.where(ii == jj, 1.0, 0.0)
    di_col = jnp.where(rowsum > 0.0, dinvr, 0.0)
    norm_adj = di_col * (adj + eye) * dj_rowb
    cuta_rowb = jax.nn.relu(alpha_rowb + 1e-7 - cut)
    s0 = norm_adj * cuta_rowb
    rs = jnp.sum(jnp.abs(s0), axis=1, keepdims=True)
    S = s0 / jnp.maximum(rs, 1e-12)
    s_ref[...] = S

    cx_ref[...] = lax.dot_general(S, x, (((0,), (0,)), ((), ())))
    t1 = jnp.dot(adj, S)
    cadj = lax.dot_general(S, t1, (((0,), (0,)), ((), ())))
    cadj_ref[...] = jnp.floor(cadj * 10000.0) / 10000.0

    iota_k = lax.broadcasted_iota(jnp.int32, (N, KPAD), 1)
    ii_k = lax.broadcasted_iota(jnp.int32, (N, KPAD), 0)
    rank_i = rank.astype(jnp.int32)
    topk_ref[...] = jnp.sum(jnp.where(iota_k == rank_i, ii_k, 0),
                            axis=0, keepdims=True)


_dense_call = functools.partial(
    pl.pallas_call,
    out_shape=[
        jax.ShapeDtypeStruct((N, D), jnp.float32),
        jax.ShapeDtypeStruct((N, N), jnp.float32),
        jax.ShapeDtypeStruct((N, N), jnp.float32),
        jax.ShapeDtypeStruct((1, KPAD), jnp.int32),
    ],
    in_specs=[
        pl.BlockSpec(memory_space=pltpu.VMEM),
        pl.BlockSpec(memory_space=pltpu.VMEM),
        pl.BlockSpec(memory_space=pltpu.VMEM),
        pl.BlockSpec(memory_space=pltpu.VMEM),
        pl.BlockSpec(memory_space=pltpu.SMEM),
    ],
    interpret=_INTERPRET,
)


def kernel(x, edge_index, edge_attr, W, b):
    src = edge_index[0]
    dst = edge_index[1]
    ew2d = edge_attr.reshape(E // _CH, _CH)
    parts = _sc_adj_call(_sc_adj_body)(src, dst, ew2d)
    a0 = parts[0].reshape(N, N)
    a1 = parts[1].reshape(N, N)
    cx, cadj, S, topk = _dense_call(_dense_body)(a0, a1, x, W, b)
    return cx, cadj, S, topk[0, :K]


# speedup vs baseline: 9.7710x; 1.5188x over previous
import functools

import jax
import jax.numpy as jnp
from jax import lax
from jax.experimental import pallas as pl
from jax.experimental.pallas import tpu as pltpu
from jax.experimental.pallas import tpu_sc as plsc

N = 1024
D = 128
E = 32768
K = 257
KPAD = 512

_INTERPRET = False

_NC = 2
_NS = 16
_L = 16
_NW = _NC * _NS
_EPW = E // _NW
_CH = 128
_NCH = _EPW // _CH
_SL = (N * N) // _NS
_ZB = 8192


def _sc_adj_body(ei_hbm, ew_hbm, out_hbm,
                 src_v, dst_v, ew_v, idx_v, z_v, adj_sh, sem):
    c = lax.axis_index("c")
    s = lax.axis_index("s")
    wid = c * _NS + s
    ebase = wid * _EPW

    pltpu.sync_copy(ei_hbm.at[0, pl.ds(ebase, _EPW)], src_v)
    pltpu.sync_copy(ei_hbm.at[1, pl.ds(ebase, _EPW)], dst_v)
    pltpu.sync_copy(ew_hbm.at[pl.ds(wid * _NCH, _NCH)], ew_v)

    def _zbody(i, carry):
        z_v[pl.ds(i * _L, _L)] = jnp.zeros((_L,), jnp.float32)
        return carry
    lax.fori_loop(0, _ZB // _L, _zbody, 0)
    for m in range(_SL // _ZB):
        pltpu.sync_copy(z_v, adj_sh.at[pl.ds(s * _SL + m * _ZB, _ZB)])

    for k in range(_EPW // _L):
        sv = src_v[pl.ds(k * _L, _L)]
        dv = dst_v[pl.ds(k * _L, _L)]
        idx_v[k // (_CH // _L), pl.ds((k % (_CH // _L)) * _L, _L)] = sv * N + dv

    plsc.subcore_barrier()

    copies = [
        pltpu.async_copy(ew_v.at[j], adj_sh.at[idx_v.at[j]], sem, add=True)
        for j in range(_NCH)
    ]
    for cp in copies:
        cp.wait()

    plsc.subcore_barrier()

    pltpu.sync_copy(adj_sh.at[pl.ds(s * _SL, _SL)],
                    out_hbm.at[c, pl.ds(s * _SL, _SL)])


_sc_adj_call = functools.partial(
    pl.kernel,
    out_type=jax.ShapeDtypeStruct((_NC, N * N), jnp.float32),
    mesh=plsc.VectorSubcoreMesh(core_axis_name="c", subcore_axis_name="s",
                                num_cores=_NC, num_subcores=_NS),
    scratch_types=[
        pltpu.VMEM((_EPW,), jnp.int32),
        pltpu.VMEM((_EPW,), jnp.int32),
        pltpu.VMEM((_NCH, _CH), jnp.float32),
        pltpu.VMEM((_NCH, _CH), jnp.int32),
        pltpu.VMEM((_ZB,), jnp.float32),
        pltpu.VMEM_SHARED((N * N,), jnp.float32),
        pltpu.SemaphoreType.DMA,
    ],
)


def _dense_body(ap_ref, x_ref, w_ref, b_ref,
                cx_ref, cadj_ref, s_ref, topk_ref):
    adj = ap_ref[0] + ap_ref[1]
    x = x_ref[...]
    W = w_ref[...]
    b = b_ref[0]
    ones_col = jnp.ones((N, 1), jnp.float32)
    HI = lax.Precision.HIGHEST

    h = jnp.dot(x, W)
    colsum = lax.dot_general(adj, ones_col, (((0,), (0,)), ((), ())),
                             precision=HI)
    dinv = lax.rsqrt(colsum + 1.0)
    vh = dinv * h
    u = lax.dot_general(adj, vh, (((0,), (0,)), ((), ())), precision=HI)
    z = dinv * u + dinv * dinv * h
    alpha = jax.nn.sigmoid(jnp.square(z + b))

    alpha_rowb = lax.dot_general(ones_col, alpha, (((1,), (1,)), ((), ())),
                                 precision=HI)
    ii = lax.broadcasted_iota(jnp.int32, (N, N), 0)
    jj = lax.broadcasted_iota(jnp.int32, (N, N), 1)
    beats = (alpha_rowb > alpha) | ((alpha_rowb == alpha) & (jj < ii))
    rank = jnp.sum(jnp.where(beats, 1.0, 0.0), axis=1, keepdims=True)
    cut = jnp.sum(jnp.where(rank == (K - 1.0), alpha, 0.0))

    rowsum = lax.dot_general(adj, ones_col, (((1,), (0,)), ((), ())),
                             precision=HI)
    dinvr = lax.rsqrt(rowsum + 1.0)
    dj_rowb = lax.dot_general(ones_col, dinvr, (((1,), (1,)), ((), ())),
                              precision=HI)
    eye = jnp.where(ii == jj, 1.0, 0.0)
    di_col = jnp.where(rowsum > 0.0, dinvr, 0.0)
    norm_adj = di_col * (adj + eye) * dj_rowb
    cuta_rowb = jax.nn.relu(alpha_rowb + 1e-7 - cut)
    s0 = norm_adj * cuta_rowb
    rs = jnp.sum(jnp.abs(s0), axis=1, keepdims=True)
    S = s0 / jnp.maximum(rs, 1e-12)
    s_ref[...] = S

    cx_ref[...] = lax.dot_general(S, x, (((0,), (0,)), ((), ())))
    t1 = jnp.dot(adj, S)
    cadj = lax.dot_general(S, t1, (((0,), (0,)), ((), ())))
    cadj_ref[...] = jnp.floor(cadj * 10000.0) / 10000.0

    iota_k = lax.broadcasted_iota(jnp.int32, (N, KPAD), 1)
    ii_k = lax.broadcasted_iota(jnp.int32, (N, KPAD), 0)
    rank_i = rank.astype(jnp.int32)
    topk_ref[...] = jnp.sum(jnp.where(iota_k == rank_i, ii_k, 0),
                            axis=0, keepdims=True)


_dense_call = functools.partial(
    pl.pallas_call,
    out_shape=[
        jax.ShapeDtypeStruct((N, D), jnp.float32),
        jax.ShapeDtypeStruct((N, N), jnp.float32),
        jax.ShapeDtypeStruct((N, N), jnp.float32),
        jax.ShapeDtypeStruct((1, KPAD), jnp.int32),
    ],
    in_specs=[
        pl.BlockSpec(memory_space=pltpu.VMEM),
        pl.BlockSpec(memory_space=pltpu.VMEM),
        pl.BlockSpec(memory_space=pltpu.VMEM),
        pl.BlockSpec(memory_space=pltpu.SMEM),
    ],
    interpret=_INTERPRET,
)


def kernel(x, edge_index, edge_attr, W, b):
    ew2d = edge_attr.reshape(E // _CH, _CH)
    parts = _sc_adj_call(_sc_adj_body)(edge_index, ew2d)
    ap = parts.reshape(_NC, N, N)
    cx, cadj, S, topk = _dense_call(_dense_body)(ap, x, W, b)
    return cx, cadj, S, topk[0, :K]


# speedup vs baseline: 11.0257x; 1.1284x over previous
import functools

import jax
import jax.numpy as jnp
from jax import lax
from jax.experimental import pallas as pl
from jax.experimental.pallas import tpu as pltpu
from jax.experimental.pallas import tpu_sc as plsc

N = 1024
D = 128
E = 32768
K = 257
KPAD = 512

_INTERPRET = False

_NC = 2
_NS = 16
_L = 16
_NW = _NC * _NS
_EPW = E // _NW
_CH = 128
_NCH = _EPW // _CH
_SL = (N * N) // _NS
_ZB = 8192


def _sc_adj_body(ei_hbm, ew_hbm, out_hbm,
                 src_v, dst_v, ew_v, idx_v, z_v, adj_sh, sem):
    c = lax.axis_index("c")
    s = lax.axis_index("s")
    wid = c * _NS + s
    ebase = wid * _EPW

    pltpu.sync_copy(ei_hbm.at[0, pl.ds(ebase, _EPW)], src_v)
    pltpu.sync_copy(ei_hbm.at[1, pl.ds(ebase, _EPW)], dst_v)
    pltpu.sync_copy(ew_hbm.at[pl.ds(wid * _NCH, _NCH)], ew_v)

    def _zbody(i, carry):
        z_v[pl.ds(i * _L, _L)] = jnp.zeros((_L,), jnp.float32)
        return carry
    lax.fori_loop(0, _ZB // _L, _zbody, 0)
    for m in range(_SL // _ZB):
        pltpu.sync_copy(z_v, adj_sh.at[pl.ds(s * _SL + m * _ZB, _ZB)])

    for k in range(_EPW // _L):
        sv = src_v[pl.ds(k * _L, _L)]
        dv = dst_v[pl.ds(k * _L, _L)]
        idx_v[k // (_CH // _L), pl.ds((k % (_CH // _L)) * _L, _L)] = sv * N + dv

    plsc.subcore_barrier()

    copies = [
        pltpu.async_copy(ew_v.at[j], adj_sh.at[idx_v.at[j]], sem, add=True)
        for j in range(_NCH)
    ]
    for cp in copies:
        cp.wait()

    plsc.subcore_barrier()

    pltpu.sync_copy(adj_sh.at[pl.ds(s * _SL, _SL)],
                    out_hbm.at[c, pl.ds(s * _SL, _SL)])


_sc_adj_call = functools.partial(
    pl.kernel,
    out_type=jax.ShapeDtypeStruct((_NC, N * N), jnp.float32),
    mesh=plsc.VectorSubcoreMesh(core_axis_name="c", subcore_axis_name="s",
                                num_cores=_NC, num_subcores=_NS),
    scratch_types=[
        pltpu.VMEM((_EPW,), jnp.int32),
        pltpu.VMEM((_EPW,), jnp.int32),
        pltpu.VMEM((_NCH, _CH), jnp.float32),
        pltpu.VMEM((_NCH, _CH), jnp.int32),
        pltpu.VMEM((_ZB,), jnp.float32),
        pltpu.VMEM_SHARED((N * N,), jnp.float32),
        pltpu.SemaphoreType.DMA,
    ],
)


def _dense_body(ap_ref, x_ref, w_ref, b_ref,
                cx_ref, cadj_ref, s_ref, topk_ref):
    adj = (ap_ref[0] + ap_ref[1]).reshape(N, N)
    x = x_ref[...]
    W = w_ref[...]
    b = b_ref[0]
    ones_col = jnp.ones((N, 1), jnp.float32)
    HI = lax.Precision.HIGHEST

    h = jnp.dot(x, W)
    colsum = lax.dot_general(adj, ones_col, (((0,), (0,)), ((), ())),
                             precision=HI)
    dinv = lax.rsqrt(colsum + 1.0)
    vh = dinv * h
    u = lax.dot_general(adj, vh, (((0,), (0,)), ((), ())), precision=HI)
    z = dinv * u + dinv * dinv * h
    alpha = jax.nn.sigmoid(jnp.square(z + b))

    alpha_rowb = lax.dot_general(ones_col, alpha, (((1,), (1,)), ((), ())),
                                 precision=HI)
    ii = lax.broadcasted_iota(jnp.int32, (N, N), 0)
    jj = lax.broadcasted_iota(jnp.int32, (N, N), 1)
    beats = (alpha_rowb > alpha) | ((alpha_rowb == alpha) & (jj < ii))
    rank = jnp.sum(jnp.where(beats, 1.0, 0.0), axis=1, keepdims=True)
    cut = jnp.sum(jnp.where(rank == (K - 1.0), alpha, 0.0))

    rowsum = lax.dot_general(adj, ones_col, (((1,), (0,)), ((), ())),
                             precision=HI)
    dinvr = lax.rsqrt(rowsum + 1.0)
    dj_rowb = lax.dot_general(ones_col, dinvr, (((1,), (1,)), ((), ())),
                              precision=HI)
    eye = jnp.where(ii == jj, 1.0, 0.0)
    di_col = jnp.where(rowsum > 0.0, dinvr, 0.0)
    norm_adj = di_col * (adj + eye) * dj_rowb
    cuta_rowb = jax.nn.relu(alpha_rowb + 1e-7 - cut)
    s0 = norm_adj * cuta_rowb
    rs = jnp.sum(jnp.abs(s0), axis=1, keepdims=True)
    S = s0 / jnp.maximum(rs, 1e-12)
    s_ref[...] = S

    cx_ref[...] = lax.dot_general(S, x, (((0,), (0,)), ((), ())))
    t1 = jnp.dot(adj, S)
    cadj = lax.dot_general(S, t1, (((0,), (0,)), ((), ())))
    cadj_ref[...] = jnp.floor(cadj * 10000.0) / 10000.0

    iota_k = lax.broadcasted_iota(jnp.int32, (N, KPAD), 1)
    ii_k = lax.broadcasted_iota(jnp.int32, (N, KPAD), 0)
    rank_i = rank.astype(jnp.int32)
    topk_ref[...] = jnp.sum(jnp.where(iota_k == rank_i, ii_k, 0),
                            axis=0, keepdims=True)


_dense_call = functools.partial(
    pl.pallas_call,
    out_shape=[
        jax.ShapeDtypeStruct((N, D), jnp.float32),
        jax.ShapeDtypeStruct((N, N), jnp.float32),
        jax.ShapeDtypeStruct((N, N), jnp.float32),
        jax.ShapeDtypeStruct((1, KPAD), jnp.int32),
    ],
    in_specs=[
        pl.BlockSpec(memory_space=pltpu.VMEM),
        pl.BlockSpec(memory_space=pltpu.VMEM),
        pl.BlockSpec(memory_space=pltpu.VMEM),
        pl.BlockSpec(memory_space=pltpu.SMEM),
    ],
    interpret=_INTERPRET,
)


def kernel(x, edge_index, edge_attr, W, b):
    ew2d = edge_attr.reshape(E // _CH, _CH)
    parts = _sc_adj_call(_sc_adj_body)(edge_index, ew2d)
    cx, cadj, S, topk = _dense_call(_dense_body)(parts, x, W, b)
    return cx, cadj, S, topk[0, :K]


# speedup vs baseline: 15.9840x; 1.4497x over previous
import functools

import jax
import jax.numpy as jnp
from jax import lax
from jax.experimental import pallas as pl
from jax.experimental.pallas import tpu as pltpu
from jax.experimental.pallas import tpu_sc as plsc

N = 1024
D = 128
E = 32768
K = 257
KPAD = 512

_INTERPRET = False

_NC = 2
_NS = 16
_L = 16
_NW = _NC * _NS
_EPW = E // _NW
_CH = 128
_NCH = _EPW // _CH
_SL = (N * N) // _NS
_ZB = 8192


def _sc_adj_body(ei_hbm, ew_hbm, out_hbm,
                 src_v, dst_v, ew_v, idx_v, z_v, adj_sh, sem):
    c = lax.axis_index("c")
    s = lax.axis_index("s")
    wid = c * _NS + s
    ebase = wid * _EPW

    pltpu.sync_copy(ei_hbm.at[0, pl.ds(ebase, _EPW)], src_v)
    pltpu.sync_copy(ei_hbm.at[1, pl.ds(ebase, _EPW)], dst_v)
    pltpu.sync_copy(ew_hbm.at[pl.ds(wid * _NCH, _NCH)], ew_v)

    def _zbody(i, carry):
        z_v[pl.ds(i * _L, _L)] = jnp.zeros((_L,), jnp.float32)
        return carry
    lax.fori_loop(0, _ZB // _L, _zbody, 0)
    for m in range(_SL // _ZB):
        pltpu.sync_copy(z_v, adj_sh.at[pl.ds(s * _SL + m * _ZB, _ZB)])

    for k in range(_EPW // _L):
        sv = src_v[pl.ds(k * _L, _L)]
        dv = dst_v[pl.ds(k * _L, _L)]
        idx_v[k // (_CH // _L), pl.ds((k % (_CH // _L)) * _L, _L)] = sv * N + dv

    plsc.subcore_barrier()

    copies = [
        pltpu.async_copy(ew_v.at[j], adj_sh.at[idx_v.at[j]], sem, add=True)
        for j in range(_NCH)
    ]
    for cp in copies:
        cp.wait()

    plsc.subcore_barrier()

    pltpu.sync_copy(adj_sh.at[pl.ds(s * _SL, _SL)],
                    out_hbm.at[c, pl.ds(s * _SL, _SL)])


_sc_adj_call = functools.partial(
    pl.kernel,
    out_type=jax.ShapeDtypeStruct((_NC, N * N), jnp.float32),
    mesh=plsc.VectorSubcoreMesh(core_axis_name="c", subcore_axis_name="s",
                                num_cores=_NC, num_subcores=_NS),
    scratch_types=[
        pltpu.VMEM((_EPW,), jnp.int32),
        pltpu.VMEM((_EPW,), jnp.int32),
        pltpu.VMEM((_NCH, _CH), jnp.float32),
        pltpu.VMEM((_NCH, _CH), jnp.int32),
        pltpu.VMEM((_ZB,), jnp.float32),
        pltpu.VMEM_SHARED((N * N,), jnp.float32),
        pltpu.SemaphoreType.DMA,
    ],
)


def _dense_body(ap_ref, x_ref, w_ref, b_ref,
                cx_ref, cadj_ref, s_ref, topk_ref):
    adj = (ap_ref[0] + ap_ref[1]).reshape(N, N)
    x = x_ref[...]
    W = w_ref[...]
    b = b_ref[0]
    HI = lax.Precision.HIGHEST
    ones_11 = jnp.ones((1, 1), jnp.float32)

    def t_col(v_row):
        return lax.dot_general(v_row, ones_11, (((0,), (0,)), ((), ())),
                               precision=HI)

    def t_row(v_col):
        return lax.dot_general(ones_11, v_col, (((1,), (1,)), ((), ())),
                               precision=HI)

    h = jnp.dot(x, W)
    colsum_row = jnp.sum(adj, axis=0, keepdims=True)
    dinv_row = lax.rsqrt(colsum_row + 1.0)
    dinv_col = t_col(dinv_row)
    vh_col = dinv_col * h
    u_row = jnp.sum(adj * vh_col, axis=0, keepdims=True)
    h_row = t_row(h)
    z_row = dinv_row * u_row + dinv_row * dinv_row * h_row
    alpha_row = jax.nn.sigmoid(jnp.square(z_row + b))
    alpha_col = t_col(alpha_row)

    ii = lax.broadcasted_iota(jnp.int32, (N, N), 0)
    jj = lax.broadcasted_iota(jnp.int32, (N, N), 1)
    beats = (alpha_row > alpha_col) | ((alpha_row == alpha_col) & (jj < ii))
    rank = jnp.sum(jnp.where(beats, 1.0, 0.0), axis=1, keepdims=True)
    cut = jnp.sum(jnp.where(rank == (K - 1.0), alpha_col, 0.0))

    rowsum_col = jnp.sum(adj, axis=1, keepdims=True)
    rowsum_row = t_row(rowsum_col)
    dinvr_col = lax.rsqrt(rowsum_col + 1.0)
    dj_row = lax.rsqrt(rowsum_row + 1.0)
    eye = jnp.where(ii == jj, 1.0, 0.0)
    di_col = jnp.where(rowsum_col > 0.0, dinvr_col, 0.0)
    cuta_row = jax.nn.relu(alpha_row + 1e-7 - cut)
    s0 = (di_col * (adj + eye)) * (dj_row * cuta_row)
    rs = jnp.sum(jnp.abs(s0), axis=1, keepdims=True)
    S = s0 / jnp.maximum(rs, 1e-12)
    s_ref[...] = S

    cx_ref[...] = lax.dot_general(S, x, (((0,), (0,)), ((), ())))
    t1 = jnp.dot(adj, S)
    cadj = lax.dot_general(S, t1, (((0,), (0,)), ((), ())))
    cadj_ref[...] = jnp.floor(cadj * 10000.0) / 10000.0

    iota_k = lax.broadcasted_iota(jnp.int32, (N, KPAD), 1)
    ii_k = lax.broadcasted_iota(jnp.int32, (N, KPAD), 0)
    rank_i = rank.astype(jnp.int32)
    topk_ref[...] = jnp.sum(jnp.where(iota_k == rank_i, ii_k, 0),
                            axis=0, keepdims=True)


_dense_call = functools.partial(
    pl.pallas_call,
    out_shape=[
        jax.ShapeDtypeStruct((N, D), jnp.float32),
        jax.ShapeDtypeStruct((N, N), jnp.float32),
        jax.ShapeDtypeStruct((N, N), jnp.float32),
        jax.ShapeDtypeStruct((1, KPAD), jnp.int32),
    ],
    in_specs=[
        pl.BlockSpec(memory_space=pltpu.VMEM),
        pl.BlockSpec(memory_space=pltpu.VMEM),
        pl.BlockSpec(memory_space=pltpu.VMEM),
        pl.BlockSpec(memory_space=pltpu.SMEM),
    ],
    interpret=_INTERPRET,
)


def kernel(x, edge_index, edge_attr, W, b):
    ew2d = edge_attr.reshape(E // _CH, _CH)
    parts = _sc_adj_call(_sc_adj_body)(edge_index, ew2d)
    cx, cadj, S, topk = _dense_call(_dense_body)(parts, x, W, b)
    return cx, cadj, S, topk[0, :K]


# speedup vs baseline: 17.0579x; 1.0672x over previous
import functools

import jax
import jax.numpy as jnp
from jax import lax
from jax.experimental import pallas as pl
from jax.experimental.pallas import tpu as pltpu
from jax.experimental.pallas import tpu_sc as plsc

N = 1024
D = 128
E = 32768
K = 257
KPAD = 512

_INTERPRET = False

_NC = 2
_NS = 16
_L = 16
_EPT = E // _NS
_CH = 128
_NCH = _EPT // _CH
_HALF = (N * N) // _NC
_SL = _HALF // _NS
_ZB = 8192


def _sc_adj_body(ei_hbm, ew_hbm, out_hbm,
                 src_v, dst_v, ew_v, idx_v, z_v, adj_sh, sem):
    c = lax.axis_index("c")
    s = lax.axis_index("s")
    ebase = s * _EPT

    pltpu.sync_copy(ei_hbm.at[0, pl.ds(ebase, _EPT)], src_v)
    pltpu.sync_copy(ei_hbm.at[1, pl.ds(ebase, _EPT)], dst_v)
    pltpu.sync_copy(ew_hbm.at[pl.ds(s * _NCH, _NCH)], ew_v)

    def _zbody(i, carry):
        z_v[pl.ds(i * _L, _L)] = jnp.zeros((_L,), jnp.float32)
        return carry
    lax.fori_loop(0, _ZB // _L, _zbody, 0)
    for m in range(_SL // _ZB):
        pltpu.sync_copy(z_v, adj_sh.at[pl.ds(s * _SL + m * _ZB, _ZB)])

    @pl.when(s == 0)
    def _():
        pltpu.sync_copy(z_v.at[pl.ds(0, _NS * _L)], adj_sh.at[pl.ds(_HALF, _NS * _L)])

    base = c * _HALF
    dump = _HALF + s * _L
    for k in range(_EPT // _L):
        sv = src_v[pl.ds(k * _L, _L)]
        dv = dst_v[pl.ds(k * _L, _L)]
        fi = sv * N + dv - base
        ok = (fi >= 0) & (fi < _HALF)
        idx_v[k // (_CH // _L), pl.ds((k % (_CH // _L)) * _L, _L)] = (
            jnp.where(ok, fi, dump))

    plsc.subcore_barrier()

    copies = [
        pltpu.async_copy(ew_v.at[j], adj_sh.at[idx_v.at[j]], sem, add=True)
        for j in range(_NCH)
    ]
    for cp in copies:
        cp.wait()

    plsc.subcore_barrier()

    pltpu.sync_copy(adj_sh.at[pl.ds(s * _SL, _SL)],
                    out_hbm.at[pl.ds(c * _HALF + s * _SL, _SL)])


_sc_adj_call = functools.partial(
    pl.kernel,
    out_type=jax.ShapeDtypeStruct((N * N,), jnp.float32),
    mesh=plsc.VectorSubcoreMesh(core_axis_name="c", subcore_axis_name="s",
                                num_cores=_NC, num_subcores=_NS),
    scratch_types=[
        pltpu.VMEM((_EPT,), jnp.int32),
        pltpu.VMEM((_EPT,), jnp.int32),
        pltpu.VMEM((_NCH, _CH), jnp.float32),
        pltpu.VMEM((_NCH, _CH), jnp.int32),
        pltpu.VMEM((_ZB,), jnp.float32),
        pltpu.VMEM_SHARED((_HALF + _NS * _L,), jnp.float32),
        pltpu.SemaphoreType.DMA,
    ],
)


def _dense_body(ap_ref, x_ref, w_ref, b_ref,
                cx_ref, cadj_ref, s_ref, topk_ref):
    adj = ap_ref[...].reshape(N, N)
    x = x_ref[...]
    W = w_ref[...]
    b = b_ref[0]
    HI = lax.Precision.HIGHEST
    ones_11 = jnp.ones((1, 1), jnp.float32)

    def t_col(v_row):
        return lax.dot_general(v_row, ones_11, (((0,), (0,)), ((), ())),
                               precision=HI)

    def t_row(v_col):
        return lax.dot_general(ones_11, v_col, (((1,), (1,)), ((), ())),
                               precision=HI)

    h = jnp.dot(x, W)
    colsum_row = jnp.sum(adj, axis=0, keepdims=True)
    dinv_row = lax.rsqrt(colsum_row + 1.0)
    dinv_col = t_col(dinv_row)
    vh_col = dinv_col * h
    u_row = jnp.sum(adj * vh_col, axis=0, keepdims=True)
    h_row = t_row(h)
    z_row = dinv_row * u_row + dinv_row * dinv_row * h_row
    alpha_row = jax.nn.sigmoid(jnp.square(z_row + b))
    alpha_col = t_col(alpha_row)

    ii = lax.broadcasted_iota(jnp.int32, (N, N), 0)
    jj = lax.broadcasted_iota(jnp.int32, (N, N), 1)
    beats = (alpha_row > alpha_col) | ((alpha_row == alpha_col) & (jj < ii))
    rank = jnp.sum(jnp.where(beats, 1.0, 0.0), axis=1, keepdims=True)
    cut = jnp.sum(jnp.where(rank == (K - 1.0), alpha_col, 0.0))

    rowsum_col = jnp.sum(adj, axis=1, keepdims=True)
    rowsum_row = t_row(rowsum_col)
    dinvr_col = lax.rsqrt(rowsum_col + 1.0)
    dj_row = lax.rsqrt(rowsum_row + 1.0)
    eye = jnp.where(ii == jj, 1.0, 0.0)
    di_col = jnp.where(rowsum_col > 0.0, dinvr_col, 0.0)
    cuta_row = jax.nn.relu(alpha_row + 1e-7 - cut)
    s0 = (di_col * (adj + eye)) * (dj_row * cuta_row)
    rs = jnp.sum(jnp.abs(s0), axis=1, keepdims=True)
    S = s0 / jnp.maximum(rs, 1e-12)
    s_ref[...] = S

    cx_ref[...] = lax.dot_general(S, x, (((0,), (0,)), ((), ())))
    t1 = jnp.dot(adj, S)
    cadj = lax.dot_general(S, t1, (((0,), (0,)), ((), ())))
    cadj_ref[...] = jnp.floor(cadj * 10000.0) / 10000.0

    iota_k = lax.broadcasted_iota(jnp.int32, (N, KPAD), 1)
    ii_k = lax.broadcasted_iota(jnp.int32, (N, KPAD), 0)
    rank_i = rank.astype(jnp.int32)
    topk_ref[...] = jnp.sum(jnp.where(iota_k == rank_i, ii_k, 0),
                            axis=0, keepdims=True)


_dense_call = functools.partial(
    pl.pallas_call,
    out_shape=[
        jax.ShapeDtypeStruct((N, D), jnp.float32),
        jax.ShapeDtypeStruct((N, N), jnp.float32),
        jax.ShapeDtypeStruct((N, N), jnp.float32),
        jax.ShapeDtypeStruct((1, KPAD), jnp.int32),
    ],
    in_specs=[
        pl.BlockSpec(memory_space=pltpu.VMEM),
        pl.BlockSpec(memory_space=pltpu.VMEM),
        pl.BlockSpec(memory_space=pltpu.VMEM),
        pl.BlockSpec(memory_space=pltpu.SMEM),
    ],
    interpret=_INTERPRET,
)


def kernel(x, edge_index, edge_attr, W, b):
    ew2d = edge_attr.reshape(E // _CH, _CH)
    parts = _sc_adj_call(_sc_adj_body)(edge_index, ew2d)
    cx, cadj, S, topk = _dense_call(_dense_body)(parts, x, W, b)
    return cx, cadj, S, topk[0, :K]


# speedup vs baseline: 17.6834x; 1.0367x over previous
import functools

import jax
import jax.numpy as jnp
from jax import lax
from jax.experimental import pallas as pl
from jax.experimental.pallas import tpu as pltpu
from jax.experimental.pallas import tpu_sc as plsc

N = 1024
D = 128
E = 32768
K = 257
KPAD = 512

_INTERPRET = False

_NC = 2
_NS = 16
_L = 16
_EPT = E // _NS
_CH = 128
_NCH = _EPT // _CH
_HALF = (N * N) // _NC
_SL = _HALF // _NS
_ZB = 8192


def _sc_adj_body(ei_hbm, ew_hbm, out_hbm,
                 src_v, dst_v, ew_v, idx_v, z_v, adj_sh, sem, zsem):
    c = lax.axis_index("c")
    s = lax.axis_index("s")
    ebase = s * _EPT

    ew_cp = pltpu.async_copy(ew_hbm.at[pl.ds(s * _NCH, _NCH)], ew_v, sem)
    pltpu.sync_copy(ei_hbm.at[0, pl.ds(ebase, _EPT)], src_v)
    pltpu.sync_copy(ei_hbm.at[1, pl.ds(ebase, _EPT)], dst_v)

    def _zbody(i, carry):
        z_v[pl.ds(i * _L, _L)] = jnp.zeros((_L,), jnp.float32)
        return carry
    lax.fori_loop(0, _ZB // _L, _zbody, 0)
    zcps = [
        pltpu.async_copy(z_v, adj_sh.at[pl.ds(s * _SL + m * _ZB, _ZB)], zsem)
        for m in range(_SL // _ZB)
    ]
    zcps.append(pltpu.async_copy(z_v.at[pl.ds(0, _L)],
                                 adj_sh.at[pl.ds(_HALF + s * _L, _L)], zsem))

    base = c * _HALF
    dump = _HALF + s * _L

    def _ibody(j, carry):
        for t in range(_CH // _L):
            sv = src_v[pl.ds(j * _CH + t * _L, _L)]
            dv = dst_v[pl.ds(j * _CH + t * _L, _L)]
            fi = sv * N + dv - base
            ok = (fi >= 0) & (fi < _HALF)
            idx_v[j, pl.ds(t * _L, _L)] = jnp.where(ok, fi, dump)
        return carry
    lax.fori_loop(0, _NCH, _ibody, 0)

    ew_cp.wait()
    for cp in zcps:
        cp.wait()
    plsc.subcore_barrier()

    copies = [
        pltpu.async_copy(ew_v.at[j], adj_sh.at[idx_v.at[j]], sem, add=True)
        for j in range(_NCH)
    ]
    for cp in copies:
        cp.wait()

    plsc.subcore_barrier()

    pltpu.sync_copy(adj_sh.at[pl.ds(s * _SL, _SL)],
                    out_hbm.at[pl.ds(c * _HALF + s * _SL, _SL)])


_sc_adj_call = functools.partial(
    pl.kernel,
    out_type=jax.ShapeDtypeStruct((N * N,), jnp.float32),
    mesh=plsc.VectorSubcoreMesh(core_axis_name="c", subcore_axis_name="s",
                                num_cores=_NC, num_subcores=_NS),
    scratch_types=[
        pltpu.VMEM((_EPT,), jnp.int32),
        pltpu.VMEM((_EPT,), jnp.int32),
        pltpu.VMEM((_NCH, _CH), jnp.float32),
        pltpu.VMEM((_NCH, _CH), jnp.int32),
        pltpu.VMEM((_ZB,), jnp.float32),
        pltpu.VMEM_SHARED((_HALF + _NS * _L,), jnp.float32),
        pltpu.SemaphoreType.DMA,
        pltpu.SemaphoreType.DMA,
    ],
)


def _dense_body(ap_ref, x_ref, w_ref, b_ref,
                cx_ref, cadj_ref, s_ref, topk_ref):
    adj = ap_ref[...].reshape(N, N)
    x = x_ref[...]
    W = w_ref[...]
    b = b_ref[0]
    HI = lax.Precision.HIGHEST
    ones_11 = jnp.ones((1, 1), jnp.float32)

    def t_col(v_row):
        return lax.dot_general(v_row, ones_11, (((0,), (0,)), ((), ())),
                               precision=HI)

    def t_row(v_col):
        return lax.dot_general(ones_11, v_col, (((1,), (1,)), ((), ())),
                               precision=HI)

    h = jnp.dot(x, W)
    colsum_row = jnp.sum(adj, axis=0, keepdims=True)
    dinv_row = lax.rsqrt(colsum_row + 1.0)
    dinv_col = t_col(dinv_row)
    vh_col = dinv_col * h
    u_row = jnp.sum(adj * vh_col, axis=0, keepdims=True)
    h_row = t_row(h)
    z_row = dinv_row * u_row + dinv_row * dinv_row * h_row
    alpha_row = jax.nn.sigmoid(jnp.square(z_row + b))
    alpha_col = t_col(alpha_row)

    ii = lax.broadcasted_iota(jnp.int32, (N, N), 0)
    jj = lax.broadcasted_iota(jnp.int32, (N, N), 1)
    beats = (alpha_row > alpha_col) | ((alpha_row == alpha_col) & (jj < ii))
    rank = jnp.sum(jnp.where(beats, 1.0, 0.0), axis=1, keepdims=True)
    cut = jnp.sum(jnp.where(rank == (K - 1.0), alpha_col, 0.0))

    rowsum_col = jnp.sum(adj, axis=1, keepdims=True)
    rowsum_row = t_row(rowsum_col)
    dinvr_col = lax.rsqrt(rowsum_col + 1.0)
    dj_row = lax.rsqrt(rowsum_row + 1.0)
    eye = jnp.where(ii == jj, 1.0, 0.0)
    di_col = jnp.where(rowsum_col > 0.0, dinvr_col, 0.0)
    cuta_row = jax.nn.relu(alpha_row + 1e-7 - cut)
    s0 = (di_col * (adj + eye)) * (dj_row * cuta_row)
    rs = jnp.sum(jnp.abs(s0), axis=1, keepdims=True)
    S = s0 / jnp.maximum(rs, 1e-12)
    s_ref[...] = S

    cx_ref[...] = lax.dot_general(S, x, (((0,), (0,)), ((), ())))
    t1 = jnp.dot(adj, S)
    cadj = lax.dot_general(S, t1, (((0,), (0,)), ((), ())))
    cadj_ref[...] = jnp.floor(cadj * 10000.0) / 10000.0

    iota_k = lax.broadcasted_iota(jnp.int32, (N, KPAD), 1)
    ii_k = lax.broadcasted_iota(jnp.int32, (N, KPAD), 0)
    rank_i = rank.astype(jnp.int32)
    topk_ref[...] = jnp.sum(jnp.where(iota_k == rank_i, ii_k, 0),
                            axis=0, keepdims=True)


_dense_call = functools.partial(
    pl.pallas_call,
    out_shape=[
        jax.ShapeDtypeStruct((N, D), jnp.float32),
        jax.ShapeDtypeStruct((N, N), jnp.float32),
        jax.ShapeDtypeStruct((N, N), jnp.float32),
        jax.ShapeDtypeStruct((1, KPAD), jnp.int32),
    ],
    in_specs=[
        pl.BlockSpec(memory_space=pltpu.VMEM),
        pl.BlockSpec(memory_space=pltpu.VMEM),
        pl.BlockSpec(memory_space=pltpu.VMEM),
        pl.BlockSpec(memory_space=pltpu.SMEM),
    ],
    interpret=_INTERPRET,
)


def kernel(x, edge_index, edge_attr, W, b):
    ew2d = edge_attr.reshape(E // _CH, _CH)
    parts = _sc_adj_call(_sc_adj_body)(edge_index, ew2d)
    cx, cadj, S, topk = _dense_call(_dense_body)(parts, x, W, b)
    return cx, cadj, S, topk[0, :K]


# speedup vs baseline: 17.6956x; 1.0007x over previous
import functools

import jax
import jax.numpy as jnp
from jax import lax
from jax.experimental import pallas as pl
from jax.experimental.pallas import tpu as pltpu
from jax.experimental.pallas import tpu_sc as plsc

N = 1024
D = 128
E = 32768
K = 257
KPAD = 512

_NC = 2
_NS = 16
_L = 16
_EPT = E // _NS
_CH = 128
_NCH = _EPT // _CH
_HALF = (N * N) // _NC
_SL = _HALF // _NS
_ZB = 8192


def _sc_adj_body(ei_hbm, ew_hbm, out_hbm,
                 src_v, dst_v, ew_v, idx_v, z_v, adj_sh, sem, zsem):
    c = lax.axis_index("c")
    s = lax.axis_index("s")
    ebase = s * _EPT

    ew_cp = pltpu.async_copy(ew_hbm.at[pl.ds(s * _NCH, _NCH)], ew_v, sem)
    pltpu.sync_copy(ei_hbm.at[0, pl.ds(ebase, _EPT)], src_v)
    pltpu.sync_copy(ei_hbm.at[1, pl.ds(ebase, _EPT)], dst_v)

    def _zbody(i, carry):
        z_v[pl.ds(i * _L, _L)] = jnp.zeros((_L,), jnp.float32)
        return carry
    lax.fori_loop(0, _ZB // _L, _zbody, 0)
    zcps = [
        pltpu.async_copy(z_v, adj_sh.at[pl.ds(s * _SL + m * _ZB, _ZB)], zsem)
        for m in range(_SL // _ZB)
    ]
    zcps.append(pltpu.async_copy(z_v.at[pl.ds(0, _L)],
                                 adj_sh.at[pl.ds(_HALF + s * _L, _L)], zsem))

    base = c * _HALF
    dump = _HALF + s * _L

    def _ibody(j, carry):
        for t in range(_CH // _L):
            sv = src_v[pl.ds(j * _CH + t * _L, _L)]
            dv = dst_v[pl.ds(j * _CH + t * _L, _L)]
            fi = sv * N + dv - base
            ok = (fi >= 0) & (fi < _HALF)
            idx_v[j, pl.ds(t * _L, _L)] = jnp.where(ok, fi, dump)
        return carry
    lax.fori_loop(0, _NCH, _ibody, 0)

    ew_cp.wait()
    for cp in zcps:
        cp.wait()
    plsc.subcore_barrier()

    copies = [
        pltpu.async_copy(ew_v.at[j], adj_sh.at[idx_v.at[j]], sem, add=True)
        for j in range(_NCH)
    ]
    for cp in copies:
        cp.wait()

    plsc.subcore_barrier()

    pltpu.sync_copy(adj_sh.at[pl.ds(s * _SL, _SL)],
                    out_hbm.at[pl.ds(c * _HALF + s * _SL, _SL)])


_sc_adj_call = functools.partial(
    pl.kernel,
    out_type=jax.ShapeDtypeStruct((N * N,), jnp.float32),
    mesh=plsc.VectorSubcoreMesh(core_axis_name="c", subcore_axis_name="s",
                                num_cores=_NC, num_subcores=_NS),
    scratch_types=[
        pltpu.VMEM((_EPT,), jnp.int32),
        pltpu.VMEM((_EPT,), jnp.int32),
        pltpu.VMEM((_NCH, _CH), jnp.float32),
        pltpu.VMEM((_NCH, _CH), jnp.int32),
        pltpu.VMEM((_ZB,), jnp.float32),
        pltpu.VMEM_SHARED((_HALF + _NS * _L,), jnp.float32),
        pltpu.SemaphoreType.DMA,
        pltpu.SemaphoreType.DMA,
    ],
)


def _dense_body(ap_ref, x_ref, w_ref, b_ref,
                cx_ref, cadj_ref, s_ref, topk_ref):
    adj = ap_ref[...].reshape(N, N)
    x = x_ref[...]
    W = w_ref[...]
    b = b_ref[0]
    HI = lax.Precision.HIGHEST
    ones_11 = jnp.ones((1, 1), jnp.float32)

    def t_col(v_row):
        return lax.dot_general(v_row, ones_11, (((0,), (0,)), ((), ())),
                               precision=HI)

    def t_row(v_col):
        return lax.dot_general(ones_11, v_col, (((1,), (1,)), ((), ())),
                               precision=HI)

    h = jnp.dot(x, W)
    colsum_row = jnp.sum(adj, axis=0, keepdims=True)
    dinv_row = lax.rsqrt(colsum_row + 1.0)
    dinv_col = t_col(dinv_row)
    vh_col = dinv_col * h
    u_row = jnp.sum(adj * vh_col, axis=0, keepdims=True)
    h_row = t_row(h)
    z_row = dinv_row * u_row + dinv_row * dinv_row * h_row
    alpha_row = jax.nn.sigmoid(jnp.square(z_row + b))
    alpha_col = t_col(alpha_row)

    ii = lax.broadcasted_iota(jnp.int32, (N, N), 0)
    jj = lax.broadcasted_iota(jnp.int32, (N, N), 1)
    beats = (alpha_row > alpha_col) | ((alpha_row == alpha_col) & (jj < ii))
    rank = jnp.sum(jnp.where(beats, 1.0, 0.0), axis=1, keepdims=True)
    cut = jnp.sum(jnp.where(rank == (K - 1.0), alpha_col, 0.0))

    rowsum_col = jnp.sum(adj, axis=1, keepdims=True)
    rowsum_row = t_row(rowsum_col)
    dinvr_col = lax.rsqrt(rowsum_col + 1.0)
    dj_row = lax.rsqrt(rowsum_row + 1.0)
    eye = jnp.where(ii == jj, 1.0, 0.0)
    di_col = jnp.where(rowsum_col > 0.0, dinvr_col, 0.0)
    cuta_row = jax.nn.relu(alpha_row + 1e-7 - cut)
    s0 = (di_col * (adj + eye)) * (dj_row * cuta_row)
    rs = jnp.sum(jnp.abs(s0), axis=1, keepdims=True)
    S = s0 / jnp.maximum(rs, 1e-12)
    s_ref[...] = S

    cx_ref[...] = lax.dot_general(S, x, (((0,), (0,)), ((), ())))
    t1 = jnp.dot(adj, S)
    cadj = lax.dot_general(S, t1, (((0,), (0,)), ((), ())))
    cadj_ref[...] = jnp.floor(cadj * 10000.0) / 10000.0

    iota_k = lax.broadcasted_iota(jnp.int32, (N, KPAD), 1)
    ii_k = lax.broadcasted_iota(jnp.int32, (N, KPAD), 0)
    rank_i = rank.astype(jnp.int32)
    topk_ref[...] = jnp.sum(jnp.where(iota_k == rank_i, ii_k, 0),
                            axis=0, keepdims=True)


_dense_call = functools.partial(
    pl.pallas_call,
    out_shape=[
        jax.ShapeDtypeStruct((N, D), jnp.float32),
        jax.ShapeDtypeStruct((N, N), jnp.float32),
        jax.ShapeDtypeStruct((N, N), jnp.float32),
        jax.ShapeDtypeStruct((1, KPAD), jnp.int32),
    ],
    in_specs=[
        pl.BlockSpec(memory_space=pltpu.VMEM),
        pl.BlockSpec(memory_space=pltpu.VMEM),
        pl.BlockSpec(memory_space=pltpu.VMEM),
        pl.BlockSpec(memory_space=pltpu.SMEM),
    ],
)


def kernel(x, edge_index, edge_attr, W, b):
    ew2d = edge_attr.reshape(E // _CH, _CH)
    parts = _sc_adj_call(_sc_adj_body)(edge_index, ew2d)
    cx, cadj, S, topk = _dense_call(_dense_body)(parts, x, W, b)
    return cx, cadj, S, topk[0, :K]
